# baseline (speedup 1.0000x reference)
import functools

import jax
import jax.numpy as jnp
from jax import lax
from jax.experimental import pallas as pl
from jax.experimental.pallas import tpu as pltpu
from jax.experimental.pallas import tpu_sc as plsc

GAMMA = 0.1
EPSILON = 0.1

NC = 2
NS = 16
NW = NC * NS
K = 128
NB = 4

N = 10000
D = 128
NP = 10240
BLK = 2048
RPT = NP // NS
RQ = 624


def _chunk_range(wid, nchunks):
    nbase = nchunks // NW
    nrem = nchunks - nbase * NW
    start = wid * nbase + jnp.minimum(wid, nrem)
    cnt = nbase + jnp.where(wid < nrem, 1, 0)
    return start, cnt


def _degree_body(idx_hbm, out_hbm, acc_sh, ones_v, ib0, ib1, ib2, ib3,
                 is0, is1, is2, is3, ss0, ss1, ss2, ss3, nchunks):
    c = lax.axis_index("c")
    s = lax.axis_index("s")
    wid = s * NC + c
    start, cnt = _chunk_range(wid, nchunks)
    ibuf = (ib0, ib1, ib2, ib3)
    isem = (is0, is1, is2, is3)
    ssem = (ss0, ss1, ss2, ss3)

    @pl.loop(0, K // 16)
    def _(i):
        ones_v[pl.ds(i * 16, 16)] = jnp.zeros((16,), jnp.float32)

    @pl.loop(0, RPT // K)
    def _(i):
        pltpu.sync_copy(ones_v, acc_sh.at[pl.ds(s * RPT + i * K, K)])

    @pl.loop(0, K // 16)
    def _(i):
        ones_v[pl.ds(i * 16, 16)] = jnp.ones((16,), jnp.float32)

    plsc.subcore_barrier()

    for p in range(2):
        @pl.when(p < cnt)
        def _():
            pltpu.async_copy(idx_hbm.at[start + p], ibuf[p], isem[p])

    nslots = (nchunks // NW + 2 + NB - 1) // NB * NB

    @pl.loop(0, nslots // NB)
    def _(i):
        for u in range(NB):
            j = i * NB + u

            @pl.when(j < cnt)
            def _():
                pltpu.make_async_copy(idx_hbm.at[start + j], ibuf[u],
                                      isem[u]).wait()

            @pl.when((j >= 1) & (j <= cnt))
            def _():
                pltpu.make_async_copy(
                    ones_v, acc_sh.at[ibuf[(u - 1) % NB].at[1]],
                    ssem[(u - 1) % NB]).wait()

            @pl.when(j < cnt)
            def _():
                pltpu.async_copy(ones_v, acc_sh.at[ibuf[u].at[1]],
                                 ssem[u], add=True)

            @pl.when(j + 2 < cnt)
            def _():
                pltpu.async_copy(idx_hbm.at[start + j + 2],
                                 ibuf[(u + 2) % NB], isem[(u + 2) % NB])

    plsc.subcore_barrier()
    pltpu.sync_copy(acc_sh.at[pl.ds(s * RPT, RPT)],
                    out_hbm.at[c].at[pl.ds(s * RPT, RPT)])


def _sc_degree(idx2):
    nchunks = idx2.shape[0]
    mesh = plsc.VectorSubcoreMesh(core_axis_name="c", subcore_axis_name="s",
                                  num_cores=NC, num_subcores=NS)
    return pl.kernel(
        functools.partial(_degree_body, nchunks=nchunks),
        out_type=jax.ShapeDtypeStruct((NC, NP), jnp.float32),
        mesh=mesh,
        scratch_types=[
            pltpu.VMEM_SHARED((NP,), jnp.float32),
            pltpu.VMEM((K,), jnp.float32),
        ] + [pltpu.VMEM((2, K), jnp.int32)] * NB
          + [pltpu.SemaphoreType.DMA] * (2 * NB),
    )(idx2)


def _segsum_body(y_hbm, idx_hbm, out_hbm, acc_sh,
                 ib0, ib1, ib2, ib3, ib4, ib5, rb0, rb1, rb2,
                 is0, is1, is2, is3, is4, is5, gs0, gs1, gs2,
                 ss0, ss1, ss2, nchunks):
    c = lax.axis_index("c")
    s = lax.axis_index("s")
    wid = s * NC + c
    start, cnt = _chunk_range(wid, nchunks)
    ibuf = (ib0, ib1, ib2, ib3, ib4, ib5)
    rbuf = (rb0, rb1, rb2)
    isem = (is0, is1, is2, is3, is4, is5)
    gsem = (gs0, gs1, gs2)
    ssem = (ss0, ss1, ss2)

    @pl.loop(0, K)
    def _(i):
        @pl.loop(0, D // 16)
        def _(j):
            rb0[i, pl.ds(j * 16, 16)] = jnp.zeros((16,), jnp.float32)

    @pl.loop(0, RQ // 104)
    def _(i):
        pltpu.sync_copy(rb0.at[pl.ds(0, 104)],
                        acc_sh.at[pl.ds(s * RQ + i * 104, 104)])

    @pl.when(s == NS - 1)
    def _():
        pltpu.sync_copy(rb0.at[pl.ds(0, 16)],
                        acc_sh.at[pl.ds(NS * RQ, 16)])

    plsc.subcore_barrier()

    for p in range(2):
        @pl.when(p < cnt)
        def _():
            pltpu.async_copy(idx_hbm.at[start + p], ibuf[p], isem[p])

    nslots = (nchunks // NW + 4 + 5) // 6 * 6

    @pl.loop(0, nslots // 6)
    def _(i):
        for u in range(6):
            j = i * 6 + u
            ur1 = (u - 1) % 3
            ur3 = (u - 3) % 3
            ui1 = (u - 1) % 6
            ui3 = (u - 3) % 6

            @pl.when(j < cnt)
            def _():
                pltpu.make_async_copy(idx_hbm.at[start + j], ibuf[u],
                                      isem[u]).wait()

            @pl.when((j >= 3) & (j <= cnt + 2))
            def _():
                pltpu.make_async_copy(rbuf[ur3], acc_sh.at[ibuf[ui3].at[1]],
                                      ssem[ur3]).wait()

            @pl.when(j < cnt)
            def _():
                pltpu.async_copy(y_hbm.at[ibuf[u].at[0]], rbuf[u % 3],
                                 gsem[u % 3])

            @pl.when((j >= 1) & (j <= cnt))
            def _():
                pltpu.make_async_copy(y_hbm.at[ibuf[ui1].at[0]], rbuf[ur1],
                                      gsem[ur1]).wait()
                pltpu.async_copy(rbuf[ur1], acc_sh.at[ibuf[ui1].at[1]],
                                 ssem[ur1], add=True)

            @pl.when(j + 2 < cnt)
            def _():
                pltpu.async_copy(idx_hbm.at[start + j + 2],
                                 ibuf[(u + 2) % 6], isem[(u + 2) % 6])

    plsc.subcore_barrier()
    pltpu.sync_copy(acc_sh.at[pl.ds(s * RQ, RQ)],
                    out_hbm.at[c].at[pl.ds(s * RQ, RQ)])

    @pl.when(s == NS - 1)
    def _():
        pltpu.sync_copy(acc_sh.at[pl.ds(NS * RQ, 16)],
                        out_hbm.at[c].at[pl.ds(NS * RQ, 16)])


def _sc_segsum(y, idx2):
    nchunks = idx2.shape[0]
    mesh = plsc.VectorSubcoreMesh(core_axis_name="c", subcore_axis_name="s",
                                  num_cores=NC, num_subcores=NS)
    return pl.kernel(
        functools.partial(_segsum_body, nchunks=nchunks),
        out_type=jax.ShapeDtypeStruct((NC, NP, D), jnp.float32),
        mesh=mesh,
        scratch_types=[pltpu.VMEM_SHARED((N, D), jnp.float32)]
        + [pltpu.VMEM((2, K), jnp.int32)] * 6
        + [pltpu.VMEM((K, D), jnp.float32)] * 3
        + [pltpu.SemaphoreType.DMA] * 12,
    )(y, idx2)


def _dinv_col(degr):
    deg = degr[0] + degr[1] + 1.0
    return lax.rsqrt(deg)[:, None]


def _tc1a_body(x_ref, phiw_ref, w_ref, xt_ref, xa_ref):
    x = x_ref[...]
    w = w_ref[...]
    xt_ref[...] = jnp.dot(x, phiw_ref[...], preferred_element_type=jnp.float32)
    xa_ref[...] = (jnp.dot(x, w.T, preferred_element_type=jnp.float32)
                   - jnp.dot(x, w, preferred_element_type=jnp.float32)
                   - GAMMA * x)


def _tc1b_body(xt_ref, xa_ref, deg_ref, b_ref, y1_ref, t1_ref):
    dinv = _dinv_col(deg_ref[...])
    xt = xt_ref[...]
    y1 = dinv * xt
    y1_ref[...] = y1
    t1_ref[...] = xa_ref[...] + dinv * y1 + b_ref[...]


def _tc2_body(x_ref, t1_ref, u_ref, deg_ref, phiw_ref, w_ref, b_ref,
              z_ref, y2_ref, t2_ref):
    x = x_ref[...]
    w = w_ref[...]
    dinv = _dinv_col(deg_ref[...])
    u1 = u_ref[0] + u_ref[1]
    x1 = x + EPSILON * jnp.tanh(t1_ref[...] + dinv * u1)
    x1l = jnp.where(x1 >= 0, x1, 0.01 * x1)
    z = x + x1l
    xt2 = jnp.dot(z, phiw_ref[...], preferred_element_type=jnp.float32)
    xa2 = (jnp.dot(z, w.T, preferred_element_type=jnp.float32)
           - jnp.dot(z, w, preferred_element_type=jnp.float32)
           - GAMMA * z)
    y2 = dinv * xt2
    z_ref[...] = z
    y2_ref[...] = y2
    t2_ref[...] = xa2 + dinv * y2 + b_ref[...]


def _tc3_body(z_ref, t2_ref, u_ref, deg_ref, out_ref):
    dinv = _dinv_col(deg_ref[...])
    u2 = u_ref[0] + u_ref[1]
    x2 = z_ref[...] + EPSILON * jnp.tanh(t2_ref[...] + dinv * u2)
    m = jnp.max(x2, axis=-1, keepdims=True)
    e = jnp.exp(x2 - m)
    out_ref[...] = e / jnp.sum(e, axis=-1, keepdims=True)


_ROW = pl.BlockSpec((BLK, D), lambda i: (i, 0))
_DEG = pl.BlockSpec((NC, BLK), lambda i: (0, i))
_UP = pl.BlockSpec((NC, BLK, D), lambda i: (0, i, 0))
_WM = pl.BlockSpec((D, D), lambda i: (0, 0))
_BV = pl.BlockSpec((1, D), lambda i: (0, 0))
_G = (NP // BLK,)
_F32 = jnp.float32


def _tc1a(xp, phiW, W):
    return pl.pallas_call(
        _tc1a_body, grid=_G,
        in_specs=[_ROW, _WM, _WM],
        out_specs=[_ROW, _ROW],
        out_shape=[jax.ShapeDtypeStruct((NP, D), _F32)] * 2,
    )(xp, phiW, W)


def _tc1b(xt, xa, deg2, b):
    return pl.pallas_call(
        _tc1b_body, grid=_G,
        in_specs=[_ROW, _ROW, _DEG, _BV],
        out_specs=[_ROW, _ROW],
        out_shape=[jax.ShapeDtypeStruct((NP, D), _F32)] * 2,
    )(xt, xa, deg2, b)


def _tc2(xp, t1, u1p, deg2, phiW, W, b):
    return pl.pallas_call(
        _tc2_body, grid=_G,
        in_specs=[_ROW, _ROW, _UP, _DEG, _WM, _WM, _BV],
        out_specs=[_ROW, _ROW, _ROW],
        out_shape=[jax.ShapeDtypeStruct((NP, D), _F32)] * 3,
    )(xp, t1, u1p, deg2, phiW, W, b)


def _tc3(z, t2, u2p, deg2):
    return pl.pallas_call(
        _tc3_body, grid=_G,
        in_specs=[_ROW, _ROW, _UP, _DEG],
        out_specs=_ROW,
        out_shape=jax.ShapeDtypeStruct((NP, D), _F32),
    )(z, t2, u2p, deg2)


def kernel(data, edge_index, W1, phiW1, b1, W2, phiW2, b2):
    E = edge_index.shape[1]
    nchunks = E // K
    idx2 = jnp.stack([edge_index[0].reshape(nchunks, K),
                      edge_index[1].reshape(nchunks, K)], axis=1)
    xp = jnp.pad(data, ((0, NP - N), (0, 0)))
    b1r = b1.reshape(1, D)
    b2r = b2.reshape(1, D)

    deg2 = _sc_degree(idx2)
    xt1, xa1 = _tc1a(xp, phiW1, W1)
    y1, t1 = _tc1b(xt1, xa1, deg2, b1r)
    u1p = _sc_segsum(y1, idx2)
    z, y2, t2 = _tc2(xp, t1, u1p, deg2, phiW2, W2, b2r)
    u2p = _sc_segsum(y2, idx2)
    return _tc3(z, t2, u2p, deg2)[:N]


# speedup vs baseline: 1.0067x; 1.0067x over previous
import functools

import jax
import jax.numpy as jnp
from jax import lax
from jax.experimental import pallas as pl
from jax.experimental.pallas import tpu as pltpu
from jax.experimental.pallas import tpu_sc as plsc

GAMMA = 0.1
EPSILON = 0.1

NC = 2
NS = 16
NW = NC * NS
K = 128
NB = 4

N = 10000
D = 128
NP = 10240
BLK = 2048
RPT = NP // NS
RQ = 624


def _chunk_range(wid, nchunks):
    nbase = nchunks // NW
    nrem = nchunks - nbase * NW
    start = wid * nbase + jnp.minimum(wid, nrem)
    cnt = nbase + jnp.where(wid < nrem, 1, 0)
    return start, cnt


def _degree_body(idx_hbm, out_hbm, acc_sh, ones_v, ib0, ib1, ib2, ib3,
                 is0, is1, is2, is3, ss0, ss1, ss2, ss3, nchunks):
    c = lax.axis_index("c")
    s = lax.axis_index("s")
    wid = s * NC + c
    start, cnt = _chunk_range(wid, nchunks)
    ibuf = (ib0, ib1, ib2, ib3)
    isem = (is0, is1, is2, is3)
    ssem = (ss0, ss1, ss2, ss3)

    @pl.loop(0, K // 16)
    def _(i):
        ones_v[pl.ds(i * 16, 16)] = jnp.zeros((16,), jnp.float32)

    @pl.loop(0, RPT // K)
    def _(i):
        pltpu.sync_copy(ones_v, acc_sh.at[pl.ds(s * RPT + i * K, K)])

    @pl.loop(0, K // 16)
    def _(i):
        ones_v[pl.ds(i * 16, 16)] = jnp.ones((16,), jnp.float32)

    plsc.subcore_barrier()

    for p in range(2):
        @pl.when(p < cnt)
        def _():
            pltpu.async_copy(idx_hbm.at[start + p], ibuf[p], isem[p])

    nslots = (nchunks // NW + 2 + NB - 1) // NB * NB

    @pl.loop(0, nslots // NB)
    def _(i):
        for u in range(NB):
            j = i * NB + u

            @pl.when(j < cnt)
            def _():
                pltpu.make_async_copy(idx_hbm.at[start + j], ibuf[u],
                                      isem[u]).wait()

            @pl.when((j >= 1) & (j <= cnt))
            def _():
                pltpu.make_async_copy(
                    ones_v, acc_sh.at[ibuf[(u - 1) % NB].at[1]],
                    ssem[(u - 1) % NB]).wait()

            @pl.when(j < cnt)
            def _():
                pltpu.async_copy(ones_v, acc_sh.at[ibuf[u].at[1]],
                                 ssem[u], add=True)

            @pl.when(j + 2 < cnt)
            def _():
                pltpu.async_copy(idx_hbm.at[start + j + 2],
                                 ibuf[(u + 2) % NB], isem[(u + 2) % NB])

    plsc.subcore_barrier()
    pltpu.sync_copy(acc_sh.at[pl.ds(s * RPT, RPT)],
                    out_hbm.at[c].at[pl.ds(s * RPT, RPT)])


def _sc_degree(idx2):
    nchunks = idx2.shape[0]
    mesh = plsc.VectorSubcoreMesh(core_axis_name="c", subcore_axis_name="s",
                                  num_cores=NC, num_subcores=NS)
    return pl.kernel(
        functools.partial(_degree_body, nchunks=nchunks),
        out_type=jax.ShapeDtypeStruct((NC, NP), jnp.float32),
        mesh=mesh,
        scratch_types=[
            pltpu.VMEM_SHARED((NP,), jnp.float32),
            pltpu.VMEM((K,), jnp.float32),
        ] + [pltpu.VMEM((2, K), jnp.int32)] * NB
          + [pltpu.SemaphoreType.DMA] * (2 * NB),
    )(idx2)


def _segsum_body(y_hbm, idx_hbm, out_hbm, acc_sh,
                 ib0, ib1, ib2, ib3, ib4, ib5, rb0, rb1, rb2,
                 is0, is1, is2, is3, is4, is5, gs0, gs1, gs2,
                 ss0, ss1, ss2, nchunks):
    c = lax.axis_index("c")
    s = lax.axis_index("s")
    wid = s * NC + c
    start, cnt = _chunk_range(wid, nchunks)
    ibuf = (ib0, ib1, ib2, ib3, ib4, ib5)
    rbuf = (rb0, rb1, rb2)
    isem = (is0, is1, is2, is3, is4, is5)
    gsem = (gs0, gs1, gs2)
    ssem = (ss0, ss1, ss2)

    for p in range(3):
        @pl.when(p < cnt)
        def _():
            pltpu.async_copy(idx_hbm.at[start + p], ibuf[p], isem[p])

    @pl.loop(0, K)
    def _(i):
        @pl.loop(0, D // 16)
        def _(j):
            rb0[i, pl.ds(j * 16, 16)] = jnp.zeros((16,), jnp.float32)

    @pl.loop(0, RQ // 104)
    def _(i):
        pltpu.sync_copy(rb0.at[pl.ds(0, 104)],
                        acc_sh.at[pl.ds(s * RQ + i * 104, 104)])

    @pl.when(s == NS - 1)
    def _():
        pltpu.sync_copy(rb0.at[pl.ds(0, 16)],
                        acc_sh.at[pl.ds(NS * RQ, 16)])

    plsc.subcore_barrier()

    nslots = (nchunks // NW + 4 + 5) // 6 * 6

    @pl.loop(0, nslots // 6)
    def _(i):
        for u in range(6):
            j = i * 6 + u
            ur1 = (u - 1) % 3
            ur3 = (u - 3) % 3
            ui1 = (u - 1) % 6
            ui3 = (u - 3) % 6

            @pl.when(j < cnt)
            def _():
                pltpu.make_async_copy(idx_hbm.at[start + j], ibuf[u],
                                      isem[u]).wait()

            @pl.when((j >= 3) & (j <= cnt + 2))
            def _():
                pltpu.make_async_copy(rbuf[ur3], acc_sh.at[ibuf[ui3].at[1]],
                                      ssem[ur3]).wait()

            @pl.when(j < cnt)
            def _():
                pltpu.async_copy(y_hbm.at[ibuf[u].at[0]], rbuf[u % 3],
                                 gsem[u % 3])

            @pl.when((j >= 1) & (j <= cnt))
            def _():
                pltpu.make_async_copy(y_hbm.at[ibuf[ui1].at[0]], rbuf[ur1],
                                      gsem[ur1]).wait()
                pltpu.async_copy(rbuf[ur1], acc_sh.at[ibuf[ui1].at[1]],
                                 ssem[ur1], add=True)

            @pl.when(j + 3 < cnt)
            def _():
                pltpu.async_copy(idx_hbm.at[start + j + 3],
                                 ibuf[(u + 3) % 6], isem[(u + 3) % 6])

    plsc.subcore_barrier()
    pltpu.sync_copy(acc_sh.at[pl.ds(s * RQ, RQ)],
                    out_hbm.at[c].at[pl.ds(s * RQ, RQ)])

    @pl.when(s == NS - 1)
    def _():
        pltpu.sync_copy(acc_sh.at[pl.ds(NS * RQ, 16)],
                        out_hbm.at[c].at[pl.ds(NS * RQ, 16)])


def _sc_segsum(y, idx2):
    nchunks = idx2.shape[0]
    mesh = plsc.VectorSubcoreMesh(core_axis_name="c", subcore_axis_name="s",
                                  num_cores=NC, num_subcores=NS)
    return pl.kernel(
        functools.partial(_segsum_body, nchunks=nchunks),
        out_type=jax.ShapeDtypeStruct((NC, NP, D), jnp.float32),
        mesh=mesh,
        scratch_types=[pltpu.VMEM_SHARED((N, D), jnp.float32)]
        + [pltpu.VMEM((2, K), jnp.int32)] * 6
        + [pltpu.VMEM((K, D), jnp.float32)] * 3
        + [pltpu.SemaphoreType.DMA] * 12,
    )(y, idx2)


def _dinv_col(degr):
    deg = degr[0] + degr[1] + 1.0
    return lax.rsqrt(deg)[:, None]


def _tc1_body(x_ref, deg_ref, phiw_ref, w_ref, b_ref, y1_ref, t1_ref):
    x = x_ref[...]
    w = w_ref[...]
    dinv = _dinv_col(deg_ref[...])
    xt = jnp.dot(x, phiw_ref[...], preferred_element_type=jnp.float32)
    xa = (jnp.dot(x, w.T, preferred_element_type=jnp.float32)
          - jnp.dot(x, w, preferred_element_type=jnp.float32)
          - GAMMA * x)
    y1 = dinv * xt
    y1_ref[...] = y1
    t1_ref[...] = xa + dinv * y1 + b_ref[...]


def _tc2_body(x_ref, t1_ref, u_ref, deg_ref, phiw_ref, w_ref, b_ref,
              z_ref, y2_ref, t2_ref):
    x = x_ref[...]
    w = w_ref[...]
    dinv = _dinv_col(deg_ref[...])
    u1 = u_ref[0] + u_ref[1]
    x1 = x + EPSILON * jnp.tanh(t1_ref[...] + dinv * u1)
    x1l = jnp.where(x1 >= 0, x1, 0.01 * x1)
    z = x + x1l
    xt2 = jnp.dot(z, phiw_ref[...], preferred_element_type=jnp.float32)
    xa2 = (jnp.dot(z, w.T, preferred_element_type=jnp.float32)
           - jnp.dot(z, w, preferred_element_type=jnp.float32)
           - GAMMA * z)
    y2 = dinv * xt2
    z_ref[...] = z
    y2_ref[...] = y2
    t2_ref[...] = xa2 + dinv * y2 + b_ref[...]


def _tc3_body(z_ref, t2_ref, u_ref, deg_ref, out_ref):
    dinv = _dinv_col(deg_ref[...])
    u2 = u_ref[0] + u_ref[1]
    x2 = z_ref[...] + EPSILON * jnp.tanh(t2_ref[...] + dinv * u2)
    m = jnp.max(x2, axis=-1, keepdims=True)
    e = jnp.exp(x2 - m)
    out_ref[...] = e / jnp.sum(e, axis=-1, keepdims=True)


_ROW = pl.BlockSpec((BLK, D), lambda i: (i, 0))
_DEG = pl.BlockSpec((NC, BLK), lambda i: (0, i))
_UP = pl.BlockSpec((NC, BLK, D), lambda i: (0, i, 0))
_WM = pl.BlockSpec((D, D), lambda i: (0, 0))
_BV = pl.BlockSpec((1, D), lambda i: (0, 0))
_G = (NP // BLK,)
_F32 = jnp.float32


def _tc1(xp, deg2, phiW, W, b):
    return pl.pallas_call(
        _tc1_body, grid=_G,
        in_specs=[_ROW, _DEG, _WM, _WM, _BV],
        out_specs=[_ROW, _ROW],
        out_shape=[jax.ShapeDtypeStruct((NP, D), _F32)] * 2,
    )(xp, deg2, phiW, W, b)


def _tc2(xp, t1, u1p, deg2, phiW, W, b):
    return pl.pallas_call(
        _tc2_body, grid=_G,
        in_specs=[_ROW, _ROW, _UP, _DEG, _WM, _WM, _BV],
        out_specs=[_ROW, _ROW, _ROW],
        out_shape=[jax.ShapeDtypeStruct((NP, D), _F32)] * 3,
    )(xp, t1, u1p, deg2, phiW, W, b)


def _tc3(z, t2, u2p, deg2):
    return pl.pallas_call(
        _tc3_body, grid=_G,
        in_specs=[_ROW, _ROW, _UP, _DEG],
        out_specs=_ROW,
        out_shape=jax.ShapeDtypeStruct((NP, D), _F32),
    )(z, t2, u2p, deg2)


def kernel(data, edge_index, W1, phiW1, b1, W2, phiW2, b2):
    E = edge_index.shape[1]
    nchunks = E // K
    idx2 = jnp.stack([edge_index[0].reshape(nchunks, K),
                      edge_index[1].reshape(nchunks, K)], axis=1)
    xp = jnp.pad(data, ((0, NP - N), (0, 0)))
    b1r = b1.reshape(1, D)
    b2r = b2.reshape(1, D)

    deg2 = _sc_degree(idx2)
    y1, t1 = _tc1(xp, deg2, phiW1, W1, b1r)
    u1p = _sc_segsum(y1, idx2)
    z, y2, t2 = _tc2(xp, t1, u1p, deg2, phiW2, W2, b2r)
    u2p = _sc_segsum(y2, idx2)
    return _tc3(z, t2, u2p, deg2)[:N]


# speedup vs baseline: 1.0108x; 1.0041x over previous
import functools

import jax
import jax.numpy as jnp
from jax import lax
from jax.experimental import pallas as pl
from jax.experimental.pallas import tpu as pltpu
from jax.experimental.pallas import tpu_sc as plsc

GAMMA = 0.1
EPSILON = 0.1

NC = 2
NS = 16
NW = NC * NS
K = 128
NB = 4

N = 10000
D = 128
NP = 10240
BLK = 2048
RPT = NP // NS
RQ = 624


def _chunk_range(wid, nchunks):
    nbase = nchunks // NW
    nrem = nchunks - nbase * NW
    start = wid * nbase + jnp.minimum(wid, nrem)
    cnt = nbase + jnp.where(wid < nrem, 1, 0)
    return start, cnt


def _degree_body(idx_hbm, out_hbm, acc_sh, ones_v, ib0, ib1, ib2, ib3,
                 is0, is1, is2, is3, ss0, ss1, ss2, ss3, nchunks):
    c = lax.axis_index("c")
    s = lax.axis_index("s")
    wid = s * NC + c
    start, cnt = _chunk_range(wid, nchunks)
    ibuf = (ib0, ib1, ib2, ib3)
    isem = (is0, is1, is2, is3)
    ssem = (ss0, ss1, ss2, ss3)

    for p in range(2):
        @pl.when(p < cnt)
        def _():
            pltpu.async_copy(idx_hbm.at[start + p], ibuf[p], isem[p])

    @pl.loop(0, K // 16)
    def _(i):
        ones_v[pl.ds(i * 16, 16)] = jnp.zeros((16,), jnp.float32)

    @pl.loop(0, RPT // K)
    def _(i):
        pltpu.sync_copy(ones_v, acc_sh.at[pl.ds(s * RPT + i * K, K)])

    @pl.loop(0, K // 16)
    def _(i):
        ones_v[pl.ds(i * 16, 16)] = jnp.ones((16,), jnp.float32)

    plsc.subcore_barrier()

    nslots = (nchunks // NW + 2 + NB - 1) // NB * NB

    @pl.loop(0, nslots // NB)
    def _(i):
        for u in range(NB):
            j = i * NB + u

            @pl.when(j < cnt)
            def _():
                pltpu.make_async_copy(idx_hbm.at[start + j], ibuf[u],
                                      isem[u]).wait()

            @pl.when((j >= 1) & (j <= cnt))
            def _():
                pltpu.make_async_copy(
                    ones_v, acc_sh.at[ibuf[(u - 1) % NB].at[1]],
                    ssem[(u - 1) % NB]).wait()

            @pl.when(j < cnt)
            def _():
                pltpu.async_copy(ones_v, acc_sh.at[ibuf[u].at[1]],
                                 ssem[u], add=True)

            @pl.when(j + 2 < cnt)
            def _():
                pltpu.async_copy(idx_hbm.at[start + j + 2],
                                 ibuf[(u + 2) % NB], isem[(u + 2) % NB])

    plsc.subcore_barrier()
    pltpu.sync_copy(acc_sh.at[pl.ds(s * RPT, RPT)],
                    out_hbm.at[c].at[pl.ds(s * RPT, RPT)])


def _sc_degree(idx2):
    nchunks = idx2.shape[0]
    mesh = plsc.VectorSubcoreMesh(core_axis_name="c", subcore_axis_name="s",
                                  num_cores=NC, num_subcores=NS)
    return pl.kernel(
        functools.partial(_degree_body, nchunks=nchunks),
        out_type=jax.ShapeDtypeStruct((NC, NP), jnp.float32),
        mesh=mesh,
        scratch_types=[
            pltpu.VMEM_SHARED((NP,), jnp.float32),
            pltpu.VMEM((K,), jnp.float32),
        ] + [pltpu.VMEM((2, K), jnp.int32)] * NB
          + [pltpu.SemaphoreType.DMA] * (2 * NB),
    )(idx2)


def _segsum_body(y_hbm, idx_hbm, out_hbm, acc_sh,
                 ib0, ib1, ib2, ib3, ib4, ib5, rb0, rb1, rb2,
                 is0, is1, is2, is3, is4, is5, gs0, gs1, gs2,
                 ss0, ss1, ss2, nchunks):
    c = lax.axis_index("c")
    s = lax.axis_index("s")
    wid = s * NC + c
    start, cnt = _chunk_range(wid, nchunks)
    ibuf = (ib0, ib1, ib2, ib3, ib4, ib5)
    rbuf = (rb0, rb1, rb2)
    isem = (is0, is1, is2, is3, is4, is5)
    gsem = (gs0, gs1, gs2)
    ssem = (ss0, ss1, ss2)

    for p in range(3):
        @pl.when(p < cnt)
        def _():
            pltpu.async_copy(idx_hbm.at[start + p], ibuf[p], isem[p])

    @pl.loop(0, K)
    def _(i):
        @pl.loop(0, D // 16)
        def _(j):
            rb0[i, pl.ds(j * 16, 16)] = jnp.zeros((16,), jnp.float32)

    @pl.loop(0, RQ // 104)
    def _(i):
        pltpu.sync_copy(rb0.at[pl.ds(0, 104)],
                        acc_sh.at[pl.ds(s * RQ + i * 104, 104)])

    @pl.when(s == NS - 1)
    def _():
        pltpu.sync_copy(rb0.at[pl.ds(0, 16)],
                        acc_sh.at[pl.ds(NS * RQ, 16)])

    plsc.subcore_barrier()

    nslots = (nchunks // NW + 4 + 5) // 6 * 6

    @pl.loop(0, nslots // 6)
    def _(i):
        for u in range(6):
            j = i * 6 + u
            ur1 = (u - 1) % 3
            ur3 = (u - 3) % 3
            ui1 = (u - 1) % 6
            ui3 = (u - 3) % 6

            @pl.when(j < cnt)
            def _():
                pltpu.make_async_copy(idx_hbm.at[start + j], ibuf[u],
                                      isem[u]).wait()

            @pl.when((j >= 3) & (j <= cnt + 2))
            def _():
                pltpu.make_async_copy(rbuf[ur3], acc_sh.at[ibuf[ui3].at[1]],
                                      ssem[ur3]).wait()

            @pl.when(j < cnt)
            def _():
                pltpu.async_copy(y_hbm.at[ibuf[u].at[0]], rbuf[u % 3],
                                 gsem[u % 3])

            @pl.when((j >= 1) & (j <= cnt))
            def _():
                pltpu.make_async_copy(y_hbm.at[ibuf[ui1].at[0]], rbuf[ur1],
                                      gsem[ur1]).wait()
                pltpu.async_copy(rbuf[ur1], acc_sh.at[ibuf[ui1].at[1]],
                                 ssem[ur1], add=True)

            @pl.when(j + 3 < cnt)
            def _():
                pltpu.async_copy(idx_hbm.at[start + j + 3],
                                 ibuf[(u + 3) % 6], isem[(u + 3) % 6])

    plsc.subcore_barrier()
    pltpu.sync_copy(acc_sh.at[pl.ds(s * RQ, RQ)],
                    out_hbm.at[c].at[pl.ds(s * RQ, RQ)])

    @pl.when(s == NS - 1)
    def _():
        pltpu.sync_copy(acc_sh.at[pl.ds(NS * RQ, 16)],
                        out_hbm.at[c].at[pl.ds(NS * RQ, 16)])


def _sc_segsum(y, idx2):
    nchunks = idx2.shape[0]
    mesh = plsc.VectorSubcoreMesh(core_axis_name="c", subcore_axis_name="s",
                                  num_cores=NC, num_subcores=NS)
    return pl.kernel(
        functools.partial(_segsum_body, nchunks=nchunks),
        out_type=jax.ShapeDtypeStruct((NC, NP, D), jnp.float32),
        mesh=mesh,
        scratch_types=[pltpu.VMEM_SHARED((N, D), jnp.float32)]
        + [pltpu.VMEM((2, K), jnp.int32)] * 6
        + [pltpu.VMEM((K, D), jnp.float32)] * 3
        + [pltpu.SemaphoreType.DMA] * 12,
    )(y, idx2)


def _dinv_col(degr):
    deg = degr[0] + degr[1] + 1.0
    return lax.rsqrt(deg)[:, None]


def _tc1_body(x_ref, deg_ref, phiw_ref, w_ref, b_ref, y1_ref, t1_ref):
    x = x_ref[...]
    w = w_ref[...]
    dinv = _dinv_col(deg_ref[...])
    xt = jnp.dot(x, phiw_ref[...], preferred_element_type=jnp.float32)
    xa = (jnp.dot(x, w.T, preferred_element_type=jnp.float32)
          - jnp.dot(x, w, preferred_element_type=jnp.float32)
          - GAMMA * x)
    y1 = dinv * xt
    y1_ref[...] = y1
    t1_ref[...] = xa + dinv * y1 + b_ref[...]


def _tc2_body(x_ref, t1_ref, u_ref, deg_ref, phiw_ref, w_ref, b_ref,
              z_ref, y2_ref, t2_ref):
    x = x_ref[...]
    w = w_ref[...]
    dinv = _dinv_col(deg_ref[...])
    u1 = u_ref[0] + u_ref[1]
    x1 = x + EPSILON * jnp.tanh(t1_ref[...] + dinv * u1)
    x1l = jnp.where(x1 >= 0, x1, 0.01 * x1)
    z = x + x1l
    xt2 = jnp.dot(z, phiw_ref[...], preferred_element_type=jnp.float32)
    xa2 = (jnp.dot(z, w.T, preferred_element_type=jnp.float32)
           - jnp.dot(z, w, preferred_element_type=jnp.float32)
           - GAMMA * z)
    y2 = dinv * xt2
    z_ref[...] = z
    y2_ref[...] = y2
    t2_ref[...] = xa2 + dinv * y2 + b_ref[...]


def _tc3_body(z_ref, t2_ref, u_ref, deg_ref, out_ref):
    dinv = _dinv_col(deg_ref[...])
    u2 = u_ref[0] + u_ref[1]
    x2 = z_ref[...] + EPSILON * jnp.tanh(t2_ref[...] + dinv * u2)
    m = jnp.max(x2, axis=-1, keepdims=True)
    e = jnp.exp(x2 - m)
    out_ref[...] = e / jnp.sum(e, axis=-1, keepdims=True)


_ROW = pl.BlockSpec((BLK, D), lambda i: (i, 0))
_DEG = pl.BlockSpec((NC, BLK), lambda i: (0, i))
_UP = pl.BlockSpec((NC, BLK, D), lambda i: (0, i, 0))
_WM = pl.BlockSpec((D, D), lambda i: (0, 0))
_BV = pl.BlockSpec((1, D), lambda i: (0, 0))
_G = (NP // BLK,)
_F32 = jnp.float32


def _tc1(xp, deg2, phiW, W, b):
    return pl.pallas_call(
        _tc1_body, grid=_G,
        in_specs=[_ROW, _DEG, _WM, _WM, _BV],
        out_specs=[_ROW, _ROW],
        out_shape=[jax.ShapeDtypeStruct((NP, D), _F32)] * 2,
    )(xp, deg2, phiW, W, b)


def _tc2(xp, t1, u1p, deg2, phiW, W, b):
    return pl.pallas_call(
        _tc2_body, grid=_G,
        in_specs=[_ROW, _ROW, _UP, _DEG, _WM, _WM, _BV],
        out_specs=[_ROW, _ROW, _ROW],
        out_shape=[jax.ShapeDtypeStruct((NP, D), _F32)] * 3,
    )(xp, t1, u1p, deg2, phiW, W, b)


def _tc3(z, t2, u2p, deg2):
    return pl.pallas_call(
        _tc3_body, grid=_G,
        in_specs=[_ROW, _ROW, _UP, _DEG],
        out_specs=_ROW,
        out_shape=jax.ShapeDtypeStruct((NP, D), _F32),
    )(z, t2, u2p, deg2)


def kernel(data, edge_index, W1, phiW1, b1, W2, phiW2, b2):
    E = edge_index.shape[1]
    nchunks = E // K
    idx2 = jnp.stack([edge_index[0].reshape(nchunks, K),
                      edge_index[1].reshape(nchunks, K)], axis=1)
    xp = jnp.pad(data, ((0, NP - N), (0, 0)))
    b1r = b1.reshape(1, D)
    b2r = b2.reshape(1, D)

    deg2 = _sc_degree(idx2)
    y1, t1 = _tc1(xp, deg2, phiW1, W1, b1r)
    u1p = _sc_segsum(y1, idx2)
    z, y2, t2 = _tc2(xp, t1, u1p, deg2, phiW2, W2, b2r)
    u2p = _sc_segsum(y2, idx2)
    return _tc3(z, t2, u2p, deg2)[:N]
